# Initial kernel scaffold; baseline (speedup 1.0000x reference)
import jax
import jax.numpy as jnp
from jax.experimental import pallas as pl


def kernel(feats, agent_feat, a2m_u, a2m_v, W1, b1, W2, g2, be2, Wa, Wf, gf, bf, gb, bb):
    raise NotImplementedError("write your pallas kernel here")


# speedup vs baseline: 10.7914x; 10.7914x over previous
import functools

import jax
import jax.numpy as jnp
from jax import lax
from jax.experimental import pallas as pl
from jax.experimental.pallas import tpu as pltpu
from jax.experimental.pallas import tpu_sc as plsc

N_MAP = 10000
N_AGT = 2048
E = 320000
D = 128

NC = 2
NS = 16
NW = NC * NS
EPW = E // NW
CHUNK = 128
NCH = EPW // CHUNK
TAIL_E = EPW - NCH * CHUNK
NIB = 4
RPT = 624
TAIL_R = N_MAP - NS * RPT

_PREC = lax.Precision.HIGHEST


def _gn(x, g, b, eps=1e-5):
    m = jnp.mean(x, axis=-1, keepdims=True)
    v = jnp.mean((x - m) ** 2, axis=-1, keepdims=True)
    return (x - m) * lax.rsqrt(v + eps) * g + b


def _agent_body(af_ref, wa_ref, out_ref):
    out_ref[...] = jnp.dot(af_ref[...], wa_ref[...],
                           preferred_element_type=jnp.float32, precision=_PREC)


def _agent_mm(agent_feat, Wa):
    return pl.pallas_call(
        _agent_body,
        out_shape=jax.ShapeDtypeStruct((N_AGT, D), jnp.float32),
    )(agent_feat, Wa)


_sc_mesh = plsc.VectorSubcoreMesh(core_axis_name="c", subcore_axis_name="s")


@functools.partial(
    pl.kernel,
    out_type=jax.ShapeDtypeStruct((NC, N_MAP, D), jnp.float32),
    mesh=_sc_mesh,
    scratch_types=[
        pltpu.VMEM_SHARED((N_MAP, D), jnp.float32),
        pltpu.VMEM((NIB, CHUNK), jnp.int32),
        pltpu.VMEM((NIB, CHUNK), jnp.int32),
        pltpu.VMEM((2, CHUNK, D), jnp.float32),
        pltpu.SemaphoreType.DMA,
        pltpu.SemaphoreType.DMA,
        pltpu.SemaphoreType.DMA,
    ],
)
def _sc_scatter(a_hbm, u_hbm, v_hbm, z_hbm, out_hbm,
                acc, u_b, v_b, rows, isem, gsem, ssem):
    c = lax.axis_index("c")
    s = lax.axis_index("s")
    wid = s * NC + c
    base = wid * EPW

    def fetch_idx(j, slot):
        pltpu.async_copy(u_hbm.at[pl.ds(base + j * CHUNK, CHUNK)],
                         u_b.at[slot], isem)
        pltpu.async_copy(v_hbm.at[pl.ds(base + j * CHUNK, CHUNK)],
                         v_b.at[slot], isem)

    def drain_idx(slot):
        pltpu.make_async_copy(u_hbm.at[pl.ds(0, CHUNK)], u_b.at[slot], isem).wait()
        pltpu.make_async_copy(v_hbm.at[pl.ds(0, CHUNK)], v_b.at[slot], isem).wait()

    def issue_gather(slot_i, h):
        pltpu.async_copy(a_hbm.at[u_b.at[slot_i]], rows.at[h], gsem)

    def drain_gather(h):
        pltpu.make_async_copy(a_hbm.at[u_b.at[0]], rows.at[h], gsem).wait()

    def issue_scatter(slot_i, h):
        pltpu.async_copy(rows.at[h], acc.at[v_b.at[slot_i]], ssem, add=True)

    def drain_scatter(h):
        pltpu.make_async_copy(rows.at[h], acc.at[v_b.at[0]], ssem).wait()

    fetch_idx(0, 0)
    fetch_idx(1, 1)
    pltpu.sync_copy(z_hbm.at[pl.ds(s * RPT, RPT)], acc.at[pl.ds(s * RPT, RPT)])

    @pl.when(s == NS - 1)
    def _():
        pltpu.sync_copy(z_hbm.at[pl.ds(NS * RPT, TAIL_R)],
                        acc.at[pl.ds(NS * RPT, TAIL_R)])

    drain_idx(0)
    issue_gather(0, 0)
    plsc.subcore_barrier()

    @pl.loop(0, NCH, step=2)
    def _chunk(j):
        for h in range(2):
            jj = j + h

            @pl.when(jj > 0)
            def _():
                drain_scatter(1 - h)

            @pl.when(jj + 1 < NCH)
            def _():
                drain_idx((jj + 1) % NIB)
                issue_gather((jj + 1) % NIB, 1 - h)

            @pl.when(jj + 2 < NCH)
            def _():
                fetch_idx(jj + 2, (jj + 2) % NIB)

            drain_gather(h)
            issue_scatter(jj % NIB, h)

    drain_scatter((NCH - 1) % 2)
    pltpu.sync_copy(u_hbm.at[pl.ds(base + NCH * CHUNK, TAIL_E)],
                    u_b.at[0, pl.ds(0, TAIL_E)])
    pltpu.sync_copy(v_hbm.at[pl.ds(base + NCH * CHUNK, TAIL_E)],
                    v_b.at[0, pl.ds(0, TAIL_E)])
    pltpu.async_copy(a_hbm.at[u_b.at[0, pl.ds(0, TAIL_E)]],
                     rows.at[0, pl.ds(0, TAIL_E)], gsem)
    pltpu.make_async_copy(a_hbm.at[u_b.at[0, pl.ds(0, TAIL_E)]],
                          rows.at[0, pl.ds(0, TAIL_E)], gsem).wait()
    pltpu.async_copy(rows.at[0, pl.ds(0, TAIL_E)],
                     acc.at[v_b.at[0, pl.ds(0, TAIL_E)]], ssem, add=True)
    pltpu.make_async_copy(rows.at[0, pl.ds(0, TAIL_E)],
                          acc.at[v_b.at[0, pl.ds(0, TAIL_E)]], ssem).wait()

    plsc.subcore_barrier()

    pltpu.sync_copy(acc.at[pl.ds(s * RPT, RPT)],
                    out_hbm.at[c, pl.ds(s * RPT, RPT)])

    @pl.when(s == NS - 1)
    def _():
        pltpu.sync_copy(acc.at[pl.ds(NS * RPT, TAIL_R)],
                        out_hbm.at[c, pl.ds(NS * RPT, TAIL_R)])


def _final_body(f_ref, p0_ref, p1_ref, w1_ref, b1_ref, w2_ref, g2_ref, be2_ref,
                wf0_ref, wf1_ref, gf_ref, bf_ref, gb_ref, bb_ref, out_ref):
    h = jnp.maximum(jnp.dot(f_ref[...], w1_ref[...],
                            preferred_element_type=jnp.float32, precision=_PREC)
                    + b1_ref[...], 0.0)
    m = jnp.maximum(_gn(jnp.dot(h, w2_ref[...],
                                preferred_element_type=jnp.float32, precision=_PREC),
                        g2_ref[...], be2_ref[...]), 0.0)
    t = jnp.maximum(_gn(p0_ref[...] + p1_ref[...], gb_ref[...], bb_ref[...]), 0.0)
    z = (jnp.dot(m, wf0_ref[...], preferred_element_type=jnp.float32, precision=_PREC)
         + jnp.dot(t, wf1_ref[...], preferred_element_type=jnp.float32, precision=_PREC))
    out_ref[...] = jnp.maximum(_gn(z, gf_ref[...], bf_ref[...]), 0.0)


_RB = 2000


def _final(feats, p0, p1, W1, b1, W2, g2, be2, Wf0, Wf1, gf, bf, gb, bb):
    row_spec = lambda w: pl.BlockSpec((_RB, w), lambda i: (i, 0))
    full_spec = lambda shape: pl.BlockSpec(shape, lambda i: (0,) * len(shape))
    return pl.pallas_call(
        _final_body,
        grid=(N_MAP // _RB,),
        in_specs=[
            row_spec(8), row_spec(D), row_spec(D),
            full_spec((8, D)), full_spec((1, D)), full_spec((D, D)),
            full_spec((1, D)), full_spec((1, D)),
            full_spec((D, D)), full_spec((D, D)),
            full_spec((1, D)), full_spec((1, D)), full_spec((1, D)), full_spec((1, D)),
        ],
        out_specs=row_spec(D),
        out_shape=jax.ShapeDtypeStruct((N_MAP, D), jnp.float32),
    )(feats, p0, p1, W1, b1, W2, g2, be2, Wf0, Wf1, gf, bf, gb, bb)


def kernel(feats, agent_feat, a2m_u, a2m_v, W1, b1, W2, g2, be2, Wa, Wf, gf, bf, gb, bb):
    A = _agent_mm(agent_feat, Wa)
    zeros = jnp.zeros((N_MAP, D), jnp.float32)
    partials = _sc_scatter(A, a2m_u, a2m_v, zeros)
    r = lambda x: x.reshape(1, D)
    return _final(feats, partials[0], partials[1], W1, r(b1), W2, r(g2), r(be2),
                  Wf[:D], Wf[D:], r(gf), r(bf), r(gb), r(bb))


# speedup vs baseline: 11.0502x; 1.0240x over previous
import functools

import jax
import jax.numpy as jnp
from jax import lax
from jax.experimental import pallas as pl
from jax.experimental.pallas import tpu as pltpu
from jax.experimental.pallas import tpu_sc as plsc

N_MAP = 10000
N_AGT = 2048
E = 320000
D = 128

NC = 2
NS = 16
NW = NC * NS
EPW = E // NW
CHUNK = 80
NCH = EPW // CHUNK
NR = 4
NIB = 6
RPT = 624
TAIL_R = N_MAP - NS * RPT

_PREC = lax.Precision.HIGHEST


def _gn(x, g, b, eps=1e-5):
    m = jnp.mean(x, axis=-1, keepdims=True)
    v = jnp.mean((x - m) ** 2, axis=-1, keepdims=True)
    return (x - m) * lax.rsqrt(v + eps) * g + b


def _agent_body(af_ref, wa_ref, out_ref):
    out_ref[...] = jnp.dot(af_ref[...], wa_ref[...],
                           preferred_element_type=jnp.float32, precision=_PREC)


def _agent_mm(agent_feat, Wa):
    return pl.pallas_call(
        _agent_body,
        out_shape=jax.ShapeDtypeStruct((N_AGT, D), jnp.float32),
    )(agent_feat, Wa)


_sc_mesh = plsc.VectorSubcoreMesh(core_axis_name="c", subcore_axis_name="s")


@functools.partial(
    pl.kernel,
    out_type=jax.ShapeDtypeStruct((NC, N_MAP, D), jnp.float32),
    mesh=_sc_mesh,
    scratch_types=[
        pltpu.VMEM_SHARED((N_MAP, D), jnp.float32),
        pltpu.VMEM((NIB, CHUNK), jnp.int32),
        pltpu.VMEM((NIB, CHUNK), jnp.int32),
        pltpu.VMEM((NR, CHUNK, D), jnp.float32),
        pltpu.SemaphoreType.DMA,
        pltpu.SemaphoreType.DMA((NR,)),
        pltpu.SemaphoreType.DMA((NR,)),
    ],
)
def _sc_scatter(a_hbm, u_hbm, v_hbm, z_hbm, out_hbm,
                acc, u_b, v_b, rows, isem, gsem, ssem):
    c = lax.axis_index("c")
    s = lax.axis_index("s")
    wid = s * NC + c
    base = wid * EPW

    def fetch_idx(j, slot):
        pltpu.async_copy(u_hbm.at[pl.ds(base + j * CHUNK, CHUNK)],
                         u_b.at[slot], isem)
        pltpu.async_copy(v_hbm.at[pl.ds(base + j * CHUNK, CHUNK)],
                         v_b.at[slot], isem)

    def drain_idx(slot):
        pltpu.make_async_copy(u_hbm.at[pl.ds(0, CHUNK)], u_b.at[slot], isem).wait()
        pltpu.make_async_copy(v_hbm.at[pl.ds(0, CHUNK)], v_b.at[slot], isem).wait()

    def issue_gather(slot_i, h):
        pltpu.async_copy(a_hbm.at[u_b.at[slot_i]], rows.at[h], gsem.at[h])

    def drain_gather(h):
        pltpu.make_async_copy(a_hbm.at[u_b.at[0]], rows.at[h], gsem.at[h]).wait()

    def issue_scatter(slot_i, h):
        pltpu.async_copy(rows.at[h], acc.at[v_b.at[slot_i]], ssem.at[h], add=True)

    def drain_scatter(h):
        pltpu.make_async_copy(rows.at[h], acc.at[v_b.at[0]], ssem.at[h]).wait()

    fetch_idx(0, 0)
    fetch_idx(1, 1)
    pltpu.sync_copy(z_hbm.at[pl.ds(s * RPT, RPT)], acc.at[pl.ds(s * RPT, RPT)])

    @pl.when(s == NS - 1)
    def _():
        pltpu.sync_copy(z_hbm.at[pl.ds(NS * RPT, TAIL_R)],
                        acc.at[pl.ds(NS * RPT, TAIL_R)])

    drain_idx(0)
    issue_gather(0, 0)
    plsc.subcore_barrier()

    @pl.loop(0, NCH)
    def _chunk(jj):
        @pl.when(jj >= NR - 1)
        def _():
            drain_scatter((jj + 1) % NR)

        @pl.when(jj + 1 < NCH)
        def _():
            drain_idx((jj + 1) % NIB)
            issue_gather((jj + 1) % NIB, (jj + 1) % NR)

        @pl.when(jj + 2 < NCH)
        def _():
            fetch_idx(jj + 2, (jj + 2) % NIB)

        drain_gather(jj % NR)
        issue_scatter(jj % NIB, jj % NR)

    for k in range(NR - 1):
        drain_scatter((NCH - (NR - 1) + k) % NR)

    plsc.subcore_barrier()

    pltpu.sync_copy(acc.at[pl.ds(s * RPT, RPT)],
                    out_hbm.at[c, pl.ds(s * RPT, RPT)])

    @pl.when(s == NS - 1)
    def _():
        pltpu.sync_copy(acc.at[pl.ds(NS * RPT, TAIL_R)],
                        out_hbm.at[c, pl.ds(NS * RPT, TAIL_R)])


def _final_body(f_ref, p0_ref, p1_ref, w1_ref, b1_ref, w2_ref, g2_ref, be2_ref,
                wf0_ref, wf1_ref, gf_ref, bf_ref, gb_ref, bb_ref, out_ref):
    h = jnp.maximum(jnp.dot(f_ref[...], w1_ref[...],
                            preferred_element_type=jnp.float32, precision=_PREC)
                    + b1_ref[...], 0.0)
    m = jnp.maximum(_gn(jnp.dot(h, w2_ref[...],
                                preferred_element_type=jnp.float32, precision=_PREC),
                        g2_ref[...], be2_ref[...]), 0.0)
    t = jnp.maximum(_gn(p0_ref[...] + p1_ref[...], gb_ref[...], bb_ref[...]), 0.0)
    z = (jnp.dot(m, wf0_ref[...], preferred_element_type=jnp.float32, precision=_PREC)
         + jnp.dot(t, wf1_ref[...], preferred_element_type=jnp.float32, precision=_PREC))
    out_ref[...] = jnp.maximum(_gn(z, gf_ref[...], bf_ref[...]), 0.0)


_RB = 2000


def _final(feats, p0, p1, W1, b1, W2, g2, be2, Wf0, Wf1, gf, bf, gb, bb):
    row_spec = lambda w: pl.BlockSpec((_RB, w), lambda i: (i, 0))
    full_spec = lambda shape: pl.BlockSpec(shape, lambda i: (0,) * len(shape))
    return pl.pallas_call(
        _final_body,
        grid=(N_MAP // _RB,),
        in_specs=[
            row_spec(8), row_spec(D), row_spec(D),
            full_spec((8, D)), full_spec((1, D)), full_spec((D, D)),
            full_spec((1, D)), full_spec((1, D)),
            full_spec((D, D)), full_spec((D, D)),
            full_spec((1, D)), full_spec((1, D)), full_spec((1, D)), full_spec((1, D)),
        ],
        out_specs=row_spec(D),
        out_shape=jax.ShapeDtypeStruct((N_MAP, D), jnp.float32),
    )(feats, p0, p1, W1, b1, W2, g2, be2, Wf0, Wf1, gf, bf, gb, bb)


def kernel(feats, agent_feat, a2m_u, a2m_v, W1, b1, W2, g2, be2, Wa, Wf, gf, bf, gb, bb):
    A = _agent_mm(agent_feat, Wa)
    zeros = jnp.zeros((N_MAP, D), jnp.float32)
    partials = _sc_scatter(A, a2m_u, a2m_v, zeros)
    r = lambda x: x.reshape(1, D)
    return _final(feats, partials[0], partials[1], W1, r(b1), W2, r(g2), r(be2),
                  Wf[:D], Wf[D:], r(gf), r(bf), r(gb), r(bb))


# speedup vs baseline: 11.7505x; 1.0634x over previous
import functools

import jax
import jax.numpy as jnp
from jax import lax
from jax.experimental import pallas as pl
from jax.experimental.pallas import tpu as pltpu
from jax.experimental.pallas import tpu_sc as plsc

N_MAP = 10000
N_AGT = 2048
E = 320000
D = 128

NC = 2
NS = 16
NW = NC * NS
EPW = E // NW
CHUNK = 80
NCH = EPW // CHUNK
NR = 4
NIB = 6
RPT = 624
TAIL_R = N_MAP - NS * RPT

_PREC = lax.Precision.HIGHEST


def _gn(x, g, b, eps=1e-5):
    m = jnp.mean(x, axis=-1, keepdims=True)
    v = jnp.mean((x - m) ** 2, axis=-1, keepdims=True)
    return (x - m) * lax.rsqrt(v + eps) * g + b


def _agent_body(af_ref, wa_ref, out_ref):
    out_ref[...] = jnp.dot(af_ref[...], wa_ref[...],
                           preferred_element_type=jnp.float32, precision=_PREC)


def _agent_mm(agent_feat, Wa):
    return pl.pallas_call(
        _agent_body,
        out_shape=jax.ShapeDtypeStruct((N_AGT, D), jnp.float32),
    )(agent_feat, Wa)


_sc_mesh = plsc.VectorSubcoreMesh(core_axis_name="c", subcore_axis_name="s")


@functools.partial(
    pl.kernel,
    out_type=jax.ShapeDtypeStruct((NC, N_MAP, D), jnp.float32),
    mesh=_sc_mesh,
    scratch_types=[
        pltpu.VMEM_SHARED((N_MAP, D), jnp.float32),
        pltpu.VMEM((NIB, CHUNK), jnp.int32),
        pltpu.VMEM((NIB, CHUNK), jnp.int32),
        pltpu.VMEM((NR, CHUNK, D), jnp.float32),
        pltpu.SemaphoreType.DMA,
        pltpu.SemaphoreType.DMA((NR,)),
        pltpu.SemaphoreType.DMA((NR,)),
    ],
)
def _sc_scatter(a_hbm, u_hbm, v_hbm, z_hbm, out_hbm,
                acc, u_b, v_b, rows, isem, gsem, ssem):
    c = lax.axis_index("c")
    s = lax.axis_index("s")
    wid = s * NC + c
    base = wid * EPW

    def fetch_idx(j, slot):
        pltpu.async_copy(u_hbm.at[pl.ds(base + j * CHUNK, CHUNK)],
                         u_b.at[slot], isem)
        pltpu.async_copy(v_hbm.at[pl.ds(base + j * CHUNK, CHUNK)],
                         v_b.at[slot], isem)

    def drain_idx(slot):
        pltpu.make_async_copy(u_hbm.at[pl.ds(0, CHUNK)], u_b.at[slot], isem).wait()
        pltpu.make_async_copy(v_hbm.at[pl.ds(0, CHUNK)], v_b.at[slot], isem).wait()

    def issue_gather(slot_i, h):
        pltpu.async_copy(a_hbm.at[u_b.at[slot_i]], rows.at[h], gsem.at[h])

    def drain_gather(h):
        pltpu.make_async_copy(a_hbm.at[u_b.at[0]], rows.at[h], gsem.at[h]).wait()

    def issue_scatter(slot_i, h):
        pltpu.async_copy(rows.at[h], acc.at[v_b.at[slot_i]], ssem.at[h], add=True)

    def drain_scatter(h):
        pltpu.make_async_copy(rows.at[h], acc.at[v_b.at[0]], ssem.at[h]).wait()

    fetch_idx(0, 0)
    fetch_idx(1, 1)
    pltpu.sync_copy(z_hbm.at[pl.ds(s * RPT, RPT)], acc.at[pl.ds(s * RPT, RPT)])

    @pl.when(s == NS - 1)
    def _():
        pltpu.sync_copy(z_hbm.at[pl.ds(NS * RPT, TAIL_R)],
                        acc.at[pl.ds(NS * RPT, TAIL_R)])

    drain_idx(0)
    issue_gather(0, 0)
    plsc.subcore_barrier()

    @pl.loop(0, NCH)
    def _chunk(jj):
        @pl.when(jj >= NR - 1)
        def _():
            drain_scatter((jj + 1) % NR)

        @pl.when(jj + 1 < NCH)
        def _():
            drain_idx((jj + 1) % NIB)
            issue_gather((jj + 1) % NIB, (jj + 1) % NR)

        @pl.when(jj + 2 < NCH)
        def _():
            fetch_idx(jj + 2, (jj + 2) % NIB)

        drain_gather(jj % NR)
        issue_scatter(jj % NIB, jj % NR)

    for k in range(NR - 1):
        drain_scatter((NCH - (NR - 1) + k) % NR)

    plsc.subcore_barrier()

    pltpu.sync_copy(acc.at[pl.ds(s * RPT, RPT)],
                    out_hbm.at[c, pl.ds(s * RPT, RPT)])

    @pl.when(s == NS - 1)
    def _():
        pltpu.sync_copy(acc.at[pl.ds(NS * RPT, TAIL_R)],
                        out_hbm.at[c, pl.ds(NS * RPT, TAIL_R)])


def _map_body(f_ref, w1_ref, b1_ref, w2_ref, g2_ref, be2_ref, m_ref):
    h = jnp.maximum(jnp.dot(f_ref[...], w1_ref[...],
                            preferred_element_type=jnp.float32, precision=_PREC)
                    + b1_ref[...], 0.0)
    m_ref[...] = jnp.maximum(_gn(jnp.dot(h, w2_ref[...],
                                         preferred_element_type=jnp.float32,
                                         precision=_PREC),
                                 g2_ref[...], be2_ref[...]), 0.0)


_RB = 2000


def _map_fc(feats, W1, b1, W2, g2, be2):
    row_spec = lambda w: pl.BlockSpec((_RB, w), lambda i: (i, 0))
    full_spec = lambda shape: pl.BlockSpec(shape, lambda i: (0,) * len(shape))
    return pl.pallas_call(
        _map_body,
        grid=(N_MAP // _RB,),
        in_specs=[
            row_spec(8), full_spec((8, D)), full_spec((1, D)), full_spec((D, D)),
            full_spec((1, D)), full_spec((1, D)),
        ],
        out_specs=row_spec(D),
        out_shape=jax.ShapeDtypeStruct((N_MAP, D), jnp.float32),
    )(feats, W1, b1, W2, g2, be2)


def _final_body(m_ref, p0_ref, p1_ref, wf0_ref, wf1_ref, gf_ref, bf_ref,
                gb_ref, bb_ref, out_ref):
    t = jnp.maximum(_gn(p0_ref[...] + p1_ref[...], gb_ref[...], bb_ref[...]), 0.0)
    z = (jnp.dot(m_ref[...], wf0_ref[...],
                 preferred_element_type=jnp.float32, precision=_PREC)
         + jnp.dot(t, wf1_ref[...], preferred_element_type=jnp.float32, precision=_PREC))
    out_ref[...] = jnp.maximum(_gn(z, gf_ref[...], bf_ref[...]), 0.0)


def _final(m, p0, p1, Wf0, Wf1, gf, bf, gb, bb):
    row_spec = lambda w: pl.BlockSpec((_RB, w), lambda i: (i, 0))
    full_spec = lambda shape: pl.BlockSpec(shape, lambda i: (0,) * len(shape))
    return pl.pallas_call(
        _final_body,
        grid=(N_MAP // _RB,),
        in_specs=[
            row_spec(D), row_spec(D), row_spec(D),
            full_spec((D, D)), full_spec((D, D)),
            full_spec((1, D)), full_spec((1, D)), full_spec((1, D)), full_spec((1, D)),
        ],
        out_specs=row_spec(D),
        out_shape=jax.ShapeDtypeStruct((N_MAP, D), jnp.float32),
    )(m, p0, p1, Wf0, Wf1, gf, bf, gb, bb)


def kernel(feats, agent_feat, a2m_u, a2m_v, W1, b1, W2, g2, be2, Wa, Wf, gf, bf, gb, bb):
    A = _agent_mm(agent_feat, Wa)
    zeros = jnp.zeros((N_MAP, D), jnp.float32)
    partials = _sc_scatter(A, a2m_u, a2m_v, zeros)
    r = lambda x: x.reshape(1, D)
    m = _map_fc(feats, W1, r(b1), W2, r(g2), r(be2))
    return _final(m, partials[0], partials[1],
                  Wf[:D], Wf[D:], r(gf), r(bf), r(gb), r(bb))


# speedup vs baseline: 12.1551x; 1.0344x over previous
import functools

import jax
import jax.numpy as jnp
from jax import lax
from jax.experimental import pallas as pl
from jax.experimental.pallas import tpu as pltpu
from jax.experimental.pallas import tpu_sc as plsc

N_MAP = 10000
N_AGT = 2048
E = 320000
D = 128

NC = 2
NS = 16
NW = NC * NS
EPW = E // NW
CHUNK = 80
NCH = EPW // CHUNK
NR = 4
NIB = 6
RPT = 624
TAIL_R = N_MAP - NS * RPT

_PREC = lax.Precision.HIGHEST


def _gn(x, g, b, eps=1e-5):
    m = jnp.mean(x, axis=-1, keepdims=True)
    v = jnp.mean((x - m) ** 2, axis=-1, keepdims=True)
    return (x - m) * lax.rsqrt(v + eps) * g + b


def _agent_body(af_ref, wa_ref, out_ref):
    out_ref[...] = jnp.dot(af_ref[...], wa_ref[...],
                           preferred_element_type=jnp.float32, precision=_PREC)


def _agent_mm(agent_feat, Wa):
    return pl.pallas_call(
        _agent_body,
        out_shape=jax.ShapeDtypeStruct((N_AGT, D), jnp.float32),
    )(agent_feat, Wa)


_sc_mesh = plsc.VectorSubcoreMesh(core_axis_name="c", subcore_axis_name="s")


@functools.partial(
    pl.kernel,
    out_type=jax.ShapeDtypeStruct((NC, N_MAP, D), jnp.float32),
    mesh=_sc_mesh,
    scratch_types=[
        pltpu.VMEM_SHARED((N_MAP, D), jnp.float32),
        pltpu.VMEM((NIB, CHUNK), jnp.int32),
        pltpu.VMEM((NIB, CHUNK), jnp.int32),
        pltpu.VMEM((NR, CHUNK, D), jnp.float32),
        pltpu.SemaphoreType.DMA,
        pltpu.SemaphoreType.DMA((NR,)),
        pltpu.SemaphoreType.DMA((NR,)),
    ],
)
def _sc_scatter(a_hbm, u_hbm, v_hbm, z_hbm, out_hbm,
                acc, u_b, v_b, rows, isem, gsem, ssem):
    c = lax.axis_index("c")
    s = lax.axis_index("s")
    wid = s * NC + c
    base = wid * EPW

    def fetch_idx(j, slot):
        pltpu.async_copy(u_hbm.at[pl.ds(base + j * CHUNK, CHUNK)],
                         u_b.at[slot], isem)
        pltpu.async_copy(v_hbm.at[pl.ds(base + j * CHUNK, CHUNK)],
                         v_b.at[slot], isem)

    def drain_idx(slot):
        pltpu.make_async_copy(u_hbm.at[pl.ds(0, CHUNK)], u_b.at[slot], isem).wait()
        pltpu.make_async_copy(v_hbm.at[pl.ds(0, CHUNK)], v_b.at[slot], isem).wait()

    def issue_gather(slot_i, h):
        pltpu.async_copy(a_hbm.at[u_b.at[slot_i]], rows.at[h], gsem.at[h])

    def drain_gather(h):
        pltpu.make_async_copy(a_hbm.at[u_b.at[0]], rows.at[h], gsem.at[h]).wait()

    def issue_scatter(slot_i, h):
        pltpu.async_copy(rows.at[h], acc.at[v_b.at[slot_i]], ssem.at[h], add=True)

    def drain_scatter(h):
        pltpu.make_async_copy(rows.at[h], acc.at[v_b.at[0]], ssem.at[h]).wait()

    fetch_idx(0, 0)
    fetch_idx(1, 1)
    pltpu.sync_copy(z_hbm.at[pl.ds(s * RPT, RPT)], acc.at[pl.ds(s * RPT, RPT)])

    @pl.when(s == NS - 1)
    def _():
        pltpu.sync_copy(z_hbm.at[pl.ds(NS * RPT, TAIL_R)],
                        acc.at[pl.ds(NS * RPT, TAIL_R)])

    drain_idx(0)
    issue_gather(0, 0)
    plsc.subcore_barrier()

    @pl.loop(0, NCH)
    def _chunk(jj):
        @pl.when(jj >= NR - 1)
        def _():
            drain_scatter((jj + 1) % NR)

        @pl.when(jj + 1 < NCH)
        def _():
            drain_idx((jj + 1) % NIB)
            issue_gather((jj + 1) % NIB, (jj + 1) % NR)

        @pl.when(jj + 2 < NCH)
        def _():
            fetch_idx(jj + 2, (jj + 2) % NIB)

        drain_gather(jj % NR)
        issue_scatter(jj % NIB, jj % NR)

    for k in range(NR - 1):
        drain_scatter((NCH - (NR - 1) + k) % NR)

    plsc.subcore_barrier()

    pltpu.sync_copy(acc.at[pl.ds(s * RPT, RPT)],
                    out_hbm.at[c, pl.ds(s * RPT, RPT)])

    @pl.when(s == NS - 1)
    def _():
        pltpu.sync_copy(acc.at[pl.ds(NS * RPT, TAIL_R)],
                        out_hbm.at[c, pl.ds(NS * RPT, TAIL_R)])


def _map_body(f_ref, w1_ref, b1_ref, w2_ref, g2_ref, be2_ref, m_ref):
    h = jnp.maximum(jnp.dot(f_ref[...], w1_ref[...],
                            preferred_element_type=jnp.float32, precision=_PREC)
                    + b1_ref[...], 0.0)
    m_ref[...] = jnp.maximum(_gn(jnp.dot(h, w2_ref[...],
                                         preferred_element_type=jnp.float32,
                                         precision=_PREC),
                                 g2_ref[...], be2_ref[...]), 0.0)


_RB = 2000


def _map_fc(feats, W1, b1, W2, g2, be2):
    row_spec = lambda w: pl.BlockSpec((_RB, w), lambda i: (i, 0))
    full_spec = lambda shape: pl.BlockSpec(shape, lambda i: (0,) * len(shape))
    return pl.pallas_call(
        _map_body,
        grid=(N_MAP // _RB,),
        in_specs=[
            row_spec(8), full_spec((8, D)), full_spec((1, D)), full_spec((D, D)),
            full_spec((1, D)), full_spec((1, D)),
        ],
        out_specs=row_spec(D),
        out_shape=jax.ShapeDtypeStruct((N_MAP, D), jnp.float32),
    )(feats, W1, b1, W2, g2, be2)


def _final_body(m_ref, p0_ref, p1_ref, wf0_ref, wf1_ref, gf_ref, bf_ref,
                gb_ref, bb_ref, out_ref):
    t = jnp.maximum(_gn(p0_ref[...] + p1_ref[...], gb_ref[...], bb_ref[...]), 0.0)
    z = (jnp.dot(m_ref[...], wf0_ref[...],
                 preferred_element_type=jnp.float32, precision=_PREC)
         + jnp.dot(t, wf1_ref[...], preferred_element_type=jnp.float32, precision=_PREC))
    out_ref[...] = jnp.maximum(_gn(z, gf_ref[...], bf_ref[...]), 0.0)


def _final(m, p0, p1, Wf0, Wf1, gf, bf, gb, bb):
    row_spec = lambda w: pl.BlockSpec((_RB, w), lambda i: (i, 0))
    full_spec = lambda shape: pl.BlockSpec(shape, lambda i: (0,) * len(shape))
    return pl.pallas_call(
        _final_body,
        grid=(N_MAP // _RB,),
        in_specs=[
            row_spec(D), row_spec(D), row_spec(D),
            full_spec((D, D)), full_spec((D, D)),
            full_spec((1, D)), full_spec((1, D)), full_spec((1, D)), full_spec((1, D)),
        ],
        out_specs=row_spec(D),
        out_shape=jax.ShapeDtypeStruct((N_MAP, D), jnp.float32),
    )(m, p0, p1, Wf0, Wf1, gf, bf, gb, bb)


REP = 4


def kernel(feats, agent_feat, a2m_u, a2m_v, W1, b1, W2, g2, be2, Wa, Wf, gf, bf, gb, bb):
    A = _agent_mm(agent_feat, Wa)
    A_rep = jnp.tile(A, (REP, 1))
    u_rep = a2m_u + (N_AGT * (jnp.arange(E, dtype=jnp.int32) % REP))
    zeros = jnp.zeros((N_MAP, D), jnp.float32)
    partials = _sc_scatter(A_rep, u_rep, a2m_v, zeros)
    r = lambda x: x.reshape(1, D)
    m = _map_fc(feats, W1, r(b1), W2, r(g2), r(be2))
    return _final(m, partials[0], partials[1],
                  Wf[:D], Wf[D:], r(gf), r(bf), r(gb), r(bb))


# speedup vs baseline: 12.6253x; 1.0387x over previous
import functools

import jax
import jax.numpy as jnp
from jax import lax
from jax.experimental import pallas as pl
from jax.experimental.pallas import tpu as pltpu
from jax.experimental.pallas import tpu_sc as plsc

N_MAP = 10000
N_AGT = 2048
E = 320000
D = 128

NC = 2
NS = 16
NW = NC * NS
EPW = E // NW
CHUNK = 80
NCH = EPW // CHUNK
NR = 4
NIB = 6
RPT = 624
TAIL_R = N_MAP - NS * RPT

_PREC = lax.Precision.HIGHEST


def _gn(x, g, b, eps=1e-5):
    m = jnp.mean(x, axis=-1, keepdims=True)
    v = jnp.mean((x - m) ** 2, axis=-1, keepdims=True)
    return (x - m) * lax.rsqrt(v + eps) * g + b


def _agent_body(af_ref, wa_ref, out_ref):
    out_ref[...] = jnp.dot(af_ref[...], wa_ref[...],
                           preferred_element_type=jnp.float32, precision=_PREC)


def _agent_mm(agent_feat, Wa):
    return pl.pallas_call(
        _agent_body,
        out_shape=jax.ShapeDtypeStruct((N_AGT, D), jnp.float32),
    )(agent_feat, Wa)


_sc_mesh = plsc.VectorSubcoreMesh(core_axis_name="c", subcore_axis_name="s")


@functools.partial(
    pl.kernel,
    out_type=jax.ShapeDtypeStruct((NC, N_MAP, D), jnp.float32),
    mesh=_sc_mesh,
    scratch_types=[
        pltpu.VMEM_SHARED((N_MAP, D), jnp.float32),
        pltpu.VMEM((NIB, CHUNK), jnp.int32),
        pltpu.VMEM((NIB, CHUNK), jnp.int32),
        pltpu.VMEM((NR, CHUNK, D), jnp.float32),
        pltpu.SemaphoreType.DMA,
        pltpu.SemaphoreType.DMA((NR,)),
        pltpu.SemaphoreType.DMA((NR,)),
    ],
)
def _sc_scatter(a_hbm, u_hbm, v_hbm, z_hbm, out_hbm,
                acc, u_b, v_b, rows, isem, gsem, ssem):
    c = lax.axis_index("c")
    s = lax.axis_index("s")
    wid = s * NC + c
    base = wid * EPW

    def fetch_idx(j, slot):
        pltpu.async_copy(u_hbm.at[pl.ds(base + j * CHUNK, CHUNK)],
                         u_b.at[slot], isem)
        pltpu.async_copy(v_hbm.at[pl.ds(base + j * CHUNK, CHUNK)],
                         v_b.at[slot], isem)

    def drain_idx(slot):
        pltpu.make_async_copy(u_hbm.at[pl.ds(0, CHUNK)], u_b.at[slot], isem).wait()
        pltpu.make_async_copy(v_hbm.at[pl.ds(0, CHUNK)], v_b.at[slot], isem).wait()

    def issue_gather(slot_i, h):
        pltpu.async_copy(a_hbm.at[u_b.at[slot_i]], rows.at[h], gsem.at[h])

    def drain_gather(h):
        pltpu.make_async_copy(a_hbm.at[u_b.at[0]], rows.at[h], gsem.at[h]).wait()

    def issue_scatter(slot_i, h):
        pltpu.async_copy(rows.at[h], acc.at[v_b.at[slot_i]], ssem.at[h], add=True)

    def drain_scatter(h):
        pltpu.make_async_copy(rows.at[h], acc.at[v_b.at[0]], ssem.at[h]).wait()

    for j in range(4):
        fetch_idx(j, j)
    pltpu.sync_copy(z_hbm.at[pl.ds(s * RPT, RPT)], acc.at[pl.ds(s * RPT, RPT)])

    @pl.when(s == NS - 1)
    def _():
        pltpu.sync_copy(z_hbm.at[pl.ds(NS * RPT, TAIL_R)],
                        acc.at[pl.ds(NS * RPT, TAIL_R)])

    for j in range(3):
        drain_idx(j)
        issue_gather(j, j)
    plsc.subcore_barrier()

    @pl.loop(0, NCH)
    def _chunk(jj):
        @pl.when(jj >= 1)
        def _():
            drain_scatter((jj - 1) % NR)

        @pl.when(jj + 3 < NCH)
        def _():
            drain_idx((jj + 3) % NIB)
            issue_gather((jj + 3) % NIB, (jj + 3) % NR)

        @pl.when(jj + 4 < NCH)
        def _():
            fetch_idx(jj + 4, (jj + 4) % NIB)

        drain_gather(jj % NR)
        issue_scatter(jj % NIB, jj % NR)

    drain_scatter((NCH - 1) % NR)

    plsc.subcore_barrier()

    pltpu.sync_copy(acc.at[pl.ds(s * RPT, RPT)],
                    out_hbm.at[c, pl.ds(s * RPT, RPT)])

    @pl.when(s == NS - 1)
    def _():
        pltpu.sync_copy(acc.at[pl.ds(NS * RPT, TAIL_R)],
                        out_hbm.at[c, pl.ds(NS * RPT, TAIL_R)])


def _map_body(f_ref, w1_ref, b1_ref, w2_ref, g2_ref, be2_ref, m_ref):
    h = jnp.maximum(jnp.dot(f_ref[...], w1_ref[...],
                            preferred_element_type=jnp.float32, precision=_PREC)
                    + b1_ref[...], 0.0)
    m_ref[...] = jnp.maximum(_gn(jnp.dot(h, w2_ref[...],
                                         preferred_element_type=jnp.float32,
                                         precision=_PREC),
                                 g2_ref[...], be2_ref[...]), 0.0)


_RB = 2000


def _map_fc(feats, W1, b1, W2, g2, be2):
    row_spec = lambda w: pl.BlockSpec((_RB, w), lambda i: (i, 0))
    full_spec = lambda shape: pl.BlockSpec(shape, lambda i: (0,) * len(shape))
    return pl.pallas_call(
        _map_body,
        grid=(N_MAP // _RB,),
        in_specs=[
            row_spec(8), full_spec((8, D)), full_spec((1, D)), full_spec((D, D)),
            full_spec((1, D)), full_spec((1, D)),
        ],
        out_specs=row_spec(D),
        out_shape=jax.ShapeDtypeStruct((N_MAP, D), jnp.float32),
    )(feats, W1, b1, W2, g2, be2)


def _final_body(m_ref, p0_ref, p1_ref, wf0_ref, wf1_ref, gf_ref, bf_ref,
                gb_ref, bb_ref, out_ref):
    t = jnp.maximum(_gn(p0_ref[...] + p1_ref[...], gb_ref[...], bb_ref[...]), 0.0)
    z = (jnp.dot(m_ref[...], wf0_ref[...],
                 preferred_element_type=jnp.float32, precision=_PREC)
         + jnp.dot(t, wf1_ref[...], preferred_element_type=jnp.float32, precision=_PREC))
    out_ref[...] = jnp.maximum(_gn(z, gf_ref[...], bf_ref[...]), 0.0)


def _final(m, p0, p1, Wf0, Wf1, gf, bf, gb, bb):
    row_spec = lambda w: pl.BlockSpec((_RB, w), lambda i: (i, 0))
    full_spec = lambda shape: pl.BlockSpec(shape, lambda i: (0,) * len(shape))
    return pl.pallas_call(
        _final_body,
        grid=(N_MAP // _RB,),
        in_specs=[
            row_spec(D), row_spec(D), row_spec(D),
            full_spec((D, D)), full_spec((D, D)),
            full_spec((1, D)), full_spec((1, D)), full_spec((1, D)), full_spec((1, D)),
        ],
        out_specs=row_spec(D),
        out_shape=jax.ShapeDtypeStruct((N_MAP, D), jnp.float32),
    )(m, p0, p1, Wf0, Wf1, gf, bf, gb, bb)


REP = 4


def kernel(feats, agent_feat, a2m_u, a2m_v, W1, b1, W2, g2, be2, Wa, Wf, gf, bf, gb, bb):
    A = _agent_mm(agent_feat, Wa)
    A_rep = jnp.tile(A, (REP, 1))
    u_rep = a2m_u + (N_AGT * (jnp.arange(E, dtype=jnp.int32) % REP))
    zeros = jnp.zeros((N_MAP, D), jnp.float32)
    partials = _sc_scatter(A_rep, u_rep, a2m_v, zeros)
    r = lambda x: x.reshape(1, D)
    m = _map_fc(feats, W1, r(b1), W2, r(g2), r(be2))
    return _final(m, partials[0], partials[1],
                  Wf[:D], Wf[D:], r(gf), r(bf), r(gb), r(bb))
